# Initial kernel scaffold; baseline (speedup 1.0000x reference)
import jax
import jax.numpy as jnp
from jax.experimental import pallas as pl


def kernel(x, edge_index, W_msg, b_msg, W_self, b_self, alpha, gamma, beta):
    raise NotImplementedError("write your pallas kernel here")


# speedup vs baseline: 4.1323x; 4.1323x over previous
import functools

import jax
import jax.numpy as jnp
from jax import lax
from jax.experimental import pallas as pl
from jax.experimental.pallas import tpu as pltpu
from jax.experimental.pallas import tpu_sc as plsc

N = 10000
D = 128
E = 320000

NC = 2
NS = 16
L = 16
HD = D // NC

CL = 128
NCHUNK = 2560
CPT = NCHUNK // NS
E_PAD = NCHUNK * CL
N_ACC = 10112
RPT = N_ACC // NS
NPADROW = N_ACC - N


def _sc_segment_sum(x0, x1, edge3):
    mesh = plsc.VectorSubcoreMesh(
        core_axis_name="c", subcore_axis_name="s", num_cores=NC, num_subcores=NS
    )

    @functools.partial(
        pl.kernel,
        out_type=[
            jax.ShapeDtypeStruct((NC, N_ACC, HD), jnp.float32),
            jax.ShapeDtypeStruct((NC, N_ACC, L), jnp.float32),
        ],
        mesh=mesh,
        compiler_params=pltpu.CompilerParams(use_tc_tiling_on_sc=False),
        scratch_types=[
            pltpu.VMEM((CPT, CL), jnp.int32),
            pltpu.VMEM((CPT, CL), jnp.int32),
            pltpu.VMEM((CL, HD), jnp.float32),
            pltpu.VMEM((CL, L), jnp.float32),
            pltpu.VMEM((CL, L), jnp.float32),
            pltpu.VMEM_SHARED((N_ACC, HD), jnp.float32),
            pltpu.VMEM_SHARED((N_ACC, L), jnp.float32),
        ],
    )
    def seg(x0_hbm, x1_hbm, e_hbm, acc_hbm, deg_hbm, src_idx, dst_idx, rows,
            ones_b, zero_b, acc_sh, deg_sh):
        c = lax.axis_index("c")
        s = lax.axis_index("s")

        zero16 = jnp.zeros((L,), jnp.float32)
        one0 = (1 - jnp.minimum(lax.iota(jnp.int32, L), 1)).astype(jnp.float32)

        def init_body(i, _):
            for j in range(HD // L):
                rows[i, pl.ds(L * j, L)] = zero16
            ones_b[i, :] = one0
            zero_b[i, :] = zero16
            return 0

        lax.fori_loop(0, CL, init_body, 0)

        base = s * RPT
        off = 0
        for nrows in (CL, CL, CL, CL, RPT - 4 * CL):
            pltpu.sync_copy(rows.at[pl.ds(0, nrows)],
                            acc_sh.at[pl.ds(base + off, nrows)])
            pltpu.sync_copy(zero_b.at[pl.ds(0, nrows)],
                            deg_sh.at[pl.ds(base + off, nrows)])
            off += nrows

        pltpu.sync_copy(e_hbm.at[0, pl.ds(s * CPT, CPT)], src_idx)
        pltpu.sync_copy(e_hbm.at[1, pl.ds(s * CPT, CPT)], dst_idx)
        plsc.subcore_barrier()

        def chunk_body(j, _):
            @pl.when(c == 0)
            def _():
                pltpu.sync_copy(x0_hbm.at[src_idx.at[j]], rows)

            @pl.when(c == 1)
            def _():
                pltpu.sync_copy(x1_hbm.at[src_idx.at[j]], rows)

            pltpu.sync_copy(rows, acc_sh.at[dst_idx.at[j]], add=True)

            @pl.when((j < CPT // 2) == (c == 0))
            def _():
                pltpu.sync_copy(ones_b, deg_sh.at[dst_idx.at[j]], add=True)

            return 0

        lax.fori_loop(0, CPT, chunk_body, 0)
        plsc.subcore_barrier()

        pltpu.sync_copy(acc_sh.at[pl.ds(base, RPT)],
                        acc_hbm.at[c, pl.ds(base, RPT)])
        pltpu.sync_copy(deg_sh.at[pl.ds(base, RPT)],
                        deg_hbm.at[c, pl.ds(base, RPT)])

    return seg(x0, x1, edge3)


def _tc_body(x_ref, acc_ref, deg_ref, wm_ref, bm_ref, ws_ref, bs_ref,
             alpha_ref, gamma_ref, beta_ref, out_ref):
    deg = deg_ref[0, :N, 0:1] + deg_ref[1, :N, 0:1]
    h = (
        jnp.dot(acc_ref[0, :N, :], wm_ref[:HD, :],
                preferred_element_type=jnp.float32)
        + jnp.dot(acc_ref[1, :N, :], wm_ref[HD:, :],
                  preferred_element_type=jnp.float32)
        + deg * bm_ref[...]
        + jnp.dot(x_ref[...], ws_ref[...], preferred_element_type=jnp.float32)
        + bs_ref[...]
    )
    h = jnp.where(h > 0.0, h, alpha_ref[0, 0] * h)
    mean = jnp.mean(h, axis=0, keepdims=True)
    var = jnp.mean((h - mean) * (h - mean), axis=0, keepdims=True)
    inv = lax.rsqrt(var + 1e-5)
    out_ref[...] = (h - mean) * inv * gamma_ref[...] + beta_ref[...]


def kernel(x, edge_index, W_msg, b_msg, W_self, b_self, alpha, gamma, beta):
    npad = E_PAD - E
    pad_dst = N + (jnp.arange(npad, dtype=jnp.int32) % NPADROW)
    pad = jnp.stack([jnp.zeros((npad,), jnp.int32), pad_dst])
    edge3 = jnp.concatenate([edge_index, pad], axis=1).reshape(2, NCHUNK, CL)

    x0 = x[:, :HD]
    x1 = x[:, HD:]
    acc, deg = _sc_segment_sum(x0, x1, edge3)

    out = pl.pallas_call(
        _tc_body,
        out_shape=jax.ShapeDtypeStruct((N, D), jnp.float32),
    )(
        x,
        acc,
        deg,
        W_msg,
        b_msg.reshape(1, D),
        W_self,
        b_self.reshape(1, D),
        alpha.reshape(1, 1),
        gamma.reshape(1, D),
        beta.reshape(1, D),
    )
    return out
